# baseline (speedup 1.0000x reference)
import functools

import jax
import jax.numpy as jnp
from jax import lax
from jax.experimental import pallas as pl
from jax.experimental.pallas import tpu as pltpu, tpu_sc as plsc

B = 1024
F = 16384
U = 4096
A = 8
L = 16
GBLK = 4


def _sc_kernel():
    info = plsc.get_sparse_core_info()
    nw = info.num_cores * info.num_subcores
    cw = U // nw
    nsteps = cw // GBLK
    mesh = plsc.VectorSubcoreMesh(core_axis_name="c", subcore_axis_name="s")

    @functools.partial(
        pl.kernel,
        out_type=jax.ShapeDtypeStruct((U, B), jnp.float32),
        mesh=mesh,
        scratch_types=dict(
            idx_v=pltpu.VMEM((nsteps, GBLK * A), jnp.int32),
            wb_v=pltpu.VMEM((cw * A * L,), jnp.float32),
            bias_v=pltpu.VMEM((cw * L,), jnp.float32),
            g0=pltpu.VMEM((GBLK * A, B), jnp.float32),
            g1=pltpu.VMEM((GBLK * A, B), jnp.float32),
            o0=pltpu.VMEM((GBLK, B), jnp.float32),
            o1=pltpu.VMEM((GBLK, B), jnp.float32),
            sg0=pltpu.SemaphoreType.DMA,
            sg1=pltpu.SemaphoreType.DMA,
            so0=pltpu.SemaphoreType.DMA,
            so1=pltpu.SemaphoreType.DMA,
        ),
        compiler_params=pltpu.CompilerParams(needs_layout_passes=False),
    )
    def k(xt_hbm, idx_hbm, wb_hbm, bias_hbm, out_hbm, *, idx_v, wb_v,
          bias_v, g0, g1, o0, o1, sg0, sg1, so0, so1):
        wid = lax.axis_index("s") * info.num_cores + lax.axis_index("c")
        cbase = wid * cw
        gb = [g0, g1]
        ob = [o0, o1]
        sg = [sg0, sg1]
        so = [so0, so1]

        pltpu.sync_copy(idx_hbm.at[pl.ds(wid * nsteps, nsteps)], idx_v)
        pltpu.sync_copy(wb_hbm.at[pl.ds(wid * cw * A * L, cw * A * L)], wb_v)
        pltpu.sync_copy(bias_hbm.at[pl.ds(wid * cw * L, cw * L)], bias_v)

        for s in range(2):
            pltpu.async_copy(xt_hbm.at[idx_v.at[s]], gb[s], sg[s])

        @pl.loop(0, nsteps, step=2)
        def _steps(s0):
            for bsel in range(2):
                s = s0 + bsel
                pltpu.make_async_copy(xt_hbm.at[idx_v.at[s]], gb[bsel],
                                      sg[bsel]).wait()

                @pl.when(s >= 2)
                def _():
                    pltpu.make_async_copy(ob[bsel], out_hbm.at[pl.ds(0, GBLK)],
                                          so[bsel]).wait()

                for bl in range(GBLK):
                    cofs = s * GBLK + bl
                    wv = [wb_v[pl.ds((cofs * A + j) * L, L)] for j in range(A)]
                    bv = bias_v[pl.ds(cofs * L, L)]

                    @plsc.parallel_loop(0, B, step=L, unroll=4)
                    def _bt(i):
                        acc = bv
                        for j in range(A):
                            acc = acc + wv[j] * gb[bsel][bl * A + j,
                                                         pl.ds(i, L)]
                        ob[bsel][bl, pl.ds(i, L)] = acc

                @pl.when(s + 2 < nsteps)
                def _():
                    pltpu.async_copy(xt_hbm.at[idx_v.at[s + 2]], gb[bsel],
                                     sg[bsel])

                pltpu.async_copy(
                    ob[bsel], out_hbm.at[pl.ds(cbase + s * GBLK, GBLK)],
                    so[bsel])

        for bsel in range(2):
            pltpu.make_async_copy(ob[bsel], out_hbm.at[pl.ds(0, GBLK)],
                                  so[bsel]).wait()

    return k


def kernel(inputs, indices, kernel, bias):
    idx = indices.astype(jnp.int32)
    rows_b = idx[:, 0].reshape(U, A)
    cols = idx[::A, 1]
    w_b = kernel.reshape(U, A).astype(jnp.float32)
    rows_by_col = jnp.zeros((U, A), jnp.int32).at[cols].set(rows_b)
    w_by_col = jnp.zeros((U, A), jnp.float32).at[cols].set(w_b)
    idx_flat = rows_by_col.reshape(U * A).reshape(-1, GBLK * A)
    wb = jnp.broadcast_to(w_by_col.reshape(U * A, 1), (U * A, L)).reshape(-1)
    bias_b = jnp.broadcast_to(
        bias.astype(jnp.float32).reshape(U, 1), (U, L)).reshape(-1)
    xt = inputs.T
    out_t = _sc_kernel()(xt, idx_flat, wb, bias_b)
    return out_t.T
